# baseline (speedup 1.0000x reference)
import functools

import jax
import jax.numpy as jnp
from jax import lax
from jax.experimental import pallas as pl
from jax.experimental.pallas import tpu as pltpu
from jax.experimental.pallas import tpu_sc as plsc

_NUM_CAMS = 16
_D = 2
_S = 200
_B = 16384
_NW = 32
_GB = 4
_UPW = 25


def _sc_lookup():
    mesh = plsc.VectorSubcoreMesh(core_axis_name="c", subcore_axis_name="s")

    @functools.partial(
        pl.kernel,
        mesh=mesh,
        out_type=jax.ShapeDtypeStruct((_S, 2 * 128, 128), jnp.float32),
        compiler_params=pltpu.CompilerParams(needs_layout_passes=False),
        scratch_types=[
            pltpu.VMEM((2 * _NUM_CAMS,), jnp.float32),
            pltpu.VMEM((2, _GB, 8, 128), jnp.int32),
            pltpu.VMEM((2, 8, 2 * _GB, 128), jnp.float32),
            pltpu.SemaphoreType.DMA,
            pltpu.SemaphoreType.DMA,
        ],
    )
    def k(tbl_hbm, idx_hbm, out_hbm, tbl_v, idx_v, out_v, sin, sout):
        w = lax.axis_index("s") * 2 + lax.axis_index("c")
        pltpu.sync_copy(tbl_hbm, tbl_v)
        tbl0 = tbl_v[pl.ds(0, 16)]
        tbl1 = tbl_v[pl.ds(16, 16)]
        u0 = w * _UPW

        def in_slice(unit):
            st = unit // 32
            b0 = (unit % 32) * _GB
            return idx_hbm.at[st, pl.ds(b0, _GB), :, :]

        def out_slice(unit):
            st = unit // 32
            b0 = (unit % 32) * _GB
            return out_hbm.at[pl.ds(st * 8, 8), pl.ds(2 * b0, 2 * _GB), :]

        pltpu.async_copy(in_slice(u0), idx_v.at[0], sin)

        def unit_step(g, _):
            buf = lax.rem(g, 2)
            pltpu.make_async_copy(in_slice(u0), idx_v.at[buf], sin).wait()

            @pl.when(g < _UPW - 1)
            def _():
                pltpu.async_copy(in_slice(u0 + g + 1), idx_v.at[1 - buf], sin)

            @pl.when(g >= 2)
            def _():
                pltpu.make_async_copy(out_v.at[buf], out_slice(u0), sout).wait()

            @plsc.parallel_loop(0, 8, 1, unroll=2)
            def ss_loop(ss):
                for bt in range(_GB):
                    for kk in range(8):
                        iv = idx_v[buf, bt, ss, pl.ds(kk * 16, 16)]
                        v0 = jnp.take_along_axis(tbl0, iv, axis=0)
                        v1 = jnp.take_along_axis(tbl1, iv, axis=0)
                        out_v[buf, ss, 2 * bt, pl.ds(kk * 16, 16)] = v0
                        out_v[buf, ss, 2 * bt + 1, pl.ds(kk * 16, 16)] = v1

            pltpu.async_copy(out_v.at[buf], out_slice(u0 + g), sout)
            return 0

        lax.fori_loop(0, _UPW, unit_step, 0)
        pltpu.make_async_copy(out_v.at[0], out_slice(u0), sout).wait()
        pltpu.make_async_copy(out_v.at[0], out_slice(u0), sout).wait()

    return k


_lookup = _sc_lookup()


@jax.jit
def kernel(i, param):
    tbl = jnp.concatenate([param[:, 0], param[:, 1]]).astype(jnp.float32)
    idx4 = i.astype(jnp.int32).reshape(128, 128, 25, 8).transpose(2, 0, 3, 1)
    out3 = _lookup(tbl, idx4)
    out = out3.reshape(_S, 128, _D, 128).transpose(1, 3, 0, 2)
    return out.reshape(_B, _S, _D)


# speedup vs baseline: 1.2981x; 1.2981x over previous
import functools

import jax
import jax.numpy as jnp
from jax import lax
from jax.experimental import pallas as pl
from jax.experimental.pallas import tpu as pltpu
from jax.experimental.pallas import tpu_sc as plsc

_NUM_CAMS = 16
_D = 2
_S = 200
_B = 16384
_NW = 32
_GB = 4
_UPW = 25


def _sc_lookup():
    mesh = plsc.VectorSubcoreMesh(core_axis_name="c", subcore_axis_name="s")

    @functools.partial(
        pl.kernel,
        mesh=mesh,
        out_type=jax.ShapeDtypeStruct((_S, 2 * 128, 128), jnp.float32),
        compiler_params=pltpu.CompilerParams(needs_layout_passes=False),
        scratch_types=[
            pltpu.VMEM((2 * _NUM_CAMS,), jnp.float32),
            pltpu.VMEM((_UPW, _GB, 8, 128), jnp.int32),
            pltpu.VMEM((2, 8, 2 * _GB, 128), jnp.float32),
            pltpu.SemaphoreType.DMA,
            pltpu.SemaphoreType.DMA,
        ],
    )
    def k(tbl_hbm, idx_hbm, out_hbm, tbl_v, idx_v, out_v, sin, sout):
        w = lax.axis_index("s") * 2 + lax.axis_index("c")
        u0 = w * _UPW
        pltpu.async_copy(idx_hbm.at[pl.ds(u0, _UPW)], idx_v, sin)
        pltpu.sync_copy(tbl_hbm, tbl_v)
        tbl0 = tbl_v[pl.ds(0, 16)]
        tbl1 = tbl_v[pl.ds(16, 16)]

        def out_slice(unit):
            st = unit // 32
            b0 = (unit % 32) * _GB
            return out_hbm.at[pl.ds(st * 8, 8), pl.ds(2 * b0, 2 * _GB), :]

        pltpu.make_async_copy(idx_hbm.at[pl.ds(u0, _UPW)], idx_v, sin).wait()

        def unit_step(g, _):
            buf = lax.rem(g, 2)

            @pl.when(g >= 2)
            def _():
                pltpu.make_async_copy(out_v.at[buf], out_slice(u0), sout).wait()

            @plsc.parallel_loop(0, 8, 1, unroll=2)
            def ss_loop(ss):
                for bt in range(_GB):
                    for kk in range(8):
                        iv = idx_v[g, bt, ss, pl.ds(kk * 16, 16)]
                        v0 = jnp.take_along_axis(tbl0, iv, axis=0)
                        v1 = jnp.take_along_axis(tbl1, iv, axis=0)
                        out_v[buf, ss, 2 * bt, pl.ds(kk * 16, 16)] = v0
                        out_v[buf, ss, 2 * bt + 1, pl.ds(kk * 16, 16)] = v1

            pltpu.async_copy(out_v.at[buf], out_slice(u0 + g), sout)
            return 0

        lax.fori_loop(0, _UPW, unit_step, 0)
        pltpu.make_async_copy(out_v.at[0], out_slice(u0), sout).wait()
        pltpu.make_async_copy(out_v.at[0], out_slice(u0), sout).wait()

    return k


_lookup = _sc_lookup()


@jax.jit
def kernel(i, param):
    tbl = jnp.concatenate([param[:, 0], param[:, 1]]).astype(jnp.float32)
    idx4 = i.astype(jnp.int32).reshape(128, 128, 25, 8).transpose(2, 0, 3, 1)
    idx_u = idx4.reshape(800, _GB, 8, 128)
    out3 = _lookup(tbl, idx_u)
    out = out3.reshape(_S, 128, _D, 128).transpose(1, 3, 0, 2)
    return out.reshape(_B, _S, _D)
